# speedup vs baseline: 30.0595x; 30.0595x over previous
import functools

import jax
import jax.numpy as jnp
from jax import lax
from jax.experimental import pallas as pl
from jax.experimental.pallas import tpu as pltpu
from jax.experimental.pallas import tpu_sc as plsc

N = 10000
E = 320000
D = 128

NC = 2
NS = 16
NW = NC * NS

NP = 10240
RPT = NP // NS

EPW = E // NW
CK = 125
DC = EPW // CK

_mesh = plsc.VectorSubcoreMesh(core_axis_name="c", subcore_axis_name="s")


@functools.partial(
    pl.kernel,
    out_type=[jax.ShapeDtypeStruct((NP,), jnp.float32) for _ in range(NC)],
    mesh=_mesh,
    scratch_types=[
        pltpu.VMEM((DC, CK), jnp.int32),
        pltpu.VMEM((128,), jnp.float32),
        pltpu.VMEM((RPT,), jnp.float32),
        pltpu.VMEM_SHARED((NP,), jnp.float32),
    ],
)
def _deg_kernel(dst_hbm, d0_hbm, d1_hbm, idx_v, ones_v, zer_v, deg_sh):
    cid = lax.axis_index("c")
    sid = lax.axis_index("s")
    wid = cid * NS + sid

    for i in range(8):
        ones_v[pl.ds(i * 16, 16)] = jnp.ones((16,), jnp.float32)
    for i in range(RPT // 16):
        zer_v[pl.ds(i * 16, 16)] = jnp.zeros((16,), jnp.float32)

    pltpu.sync_copy(zer_v, deg_sh.at[pl.ds(sid * RPT, RPT)])
    pltpu.sync_copy(dst_hbm.at[wid], idx_v)
    plsc.subcore_barrier()

    def body(j, carry):
        pltpu.sync_copy(ones_v.at[pl.ds(0, CK)], deg_sh.at[idx_v.at[j]], add=True)
        return carry

    lax.fori_loop(0, DC, body, 0)
    plsc.subcore_barrier()

    sl = pl.ds(sid * RPT, RPT)

    @pl.when(cid == 0)
    def _():
        pltpu.sync_copy(deg_sh.at[sl], d0_hbm.at[sl])

    @pl.when(cid == 1)
    def _():
        pltpu.sync_copy(deg_sh.at[sl], d1_hbm.at[sl])


def _mm_body(x_ref, w_ref, d0_ref, d1_ref, h2_ref):
    deg = d0_ref[...] + d1_ref[...] + 1.0
    dis = lax.rsqrt(deg)
    h = jnp.dot(x_ref[...], w_ref[...], preferred_element_type=jnp.float32)
    h2_ref[...] = h * dis


_BN = 400


def _matmul_scale(x, W, d0, d1):
    grid = (N // _BN,)
    return pl.pallas_call(
        _mm_body,
        grid=grid,
        in_specs=[
            pl.BlockSpec((_BN, D), lambda i: (i, 0)),
            pl.BlockSpec((D, D), lambda i: (0, 0)),
            pl.BlockSpec((_BN, 1), lambda i: (i, 0)),
            pl.BlockSpec((_BN, 1), lambda i: (i, 0)),
        ],
        out_specs=pl.BlockSpec((_BN, D), lambda i: (i, 0)),
        out_shape=jax.ShapeDtypeStruct((N, D), jnp.float32),
    )(x, W, d0, d1)


@functools.partial(
    pl.kernel,
    out_type=jax.ShapeDtypeStruct((NC, NP, D), jnp.float32),
    mesh=_mesh,
    scratch_types=[
        pltpu.VMEM((DC, CK), jnp.int32),
        pltpu.VMEM((DC, CK), jnp.int32),
        pltpu.VMEM((CK, D), jnp.float32),
        pltpu.VMEM((64, D), jnp.float32),
        pltpu.VMEM_SHARED((NP, D), jnp.float32),
        pltpu.SemaphoreType.DMA,
    ],
)
def _agg_kernel(src_hbm, dst_hbm, h2_hbm, t_hbm, si_v, di_v, rows_v, zer_v,
                t_sh, sem):
    cid = lax.axis_index("c")
    sid = lax.axis_index("s")
    wid = cid * NS + sid

    def zrow(r, carry):
        for c in range(D // 16):
            zer_v[r, pl.ds(c * 16, 16)] = jnp.zeros((16,), jnp.float32)
        return carry

    lax.fori_loop(0, 64, zrow, 0)

    def zcopy(k, carry):
        pltpu.sync_copy(zer_v, t_sh.at[pl.ds(sid * RPT + k * 64, 64)])
        return carry

    lax.fori_loop(0, RPT // 64, zcopy, 0)

    pltpu.sync_copy(src_hbm.at[wid], si_v)
    pltpu.sync_copy(dst_hbm.at[wid], di_v)
    plsc.subcore_barrier()

    def body(j, carry):
        pltpu.async_copy(h2_hbm.at[si_v.at[j]], rows_v, sem).wait()
        pltpu.sync_copy(rows_v, t_sh.at[di_v.at[j]], add=True)
        return carry

    lax.fori_loop(0, DC, body, 0)
    plsc.subcore_barrier()

    def ocopy(k, carry):
        sl = pl.ds(sid * RPT + k * 64, 64)
        pltpu.sync_copy(t_sh.at[sl], t_hbm.at[cid, sl])
        return carry

    lax.fori_loop(0, RPT // 64, ocopy, 0)


def _fin_body(t_ref, h2_ref, d0_ref, d1_ref, b_ref, o_ref):
    deg = d0_ref[...] + d1_ref[...] + 1.0
    dis = lax.rsqrt(deg)
    s = t_ref[0] + t_ref[1] + h2_ref[...]
    o_ref[...] = dis * s + b_ref[...]


def _finalize(t, h2, d0, d1, b):
    grid = (N // _BN,)
    return pl.pallas_call(
        _fin_body,
        grid=grid,
        in_specs=[
            pl.BlockSpec((NC, _BN, D), lambda i: (0, i, 0)),
            pl.BlockSpec((_BN, D), lambda i: (i, 0)),
            pl.BlockSpec((_BN, 1), lambda i: (i, 0)),
            pl.BlockSpec((_BN, 1), lambda i: (i, 0)),
            pl.BlockSpec((1, D), lambda i: (0, 0)),
        ],
        out_specs=pl.BlockSpec((_BN, D), lambda i: (i, 0)),
        out_shape=jax.ShapeDtypeStruct((N, D), jnp.float32),
    )(t, h2, d0, d1, b)


def kernel(x, edge_index, W, b):
    src = edge_index[0].reshape(NW, DC, CK)
    dst = edge_index[1].reshape(NW, DC, CK)
    d0, d1 = _deg_kernel(dst)
    d0c = d0.reshape(NP, 1)
    d1c = d1.reshape(NP, 1)
    h2 = _matmul_scale(x, W, d0c, d1c)
    t = _agg_kernel(src, dst, h2)
    return _finalize(t, h2, d0c, d1c, b.reshape(1, D))


# speedup vs baseline: 32.2308x; 1.0722x over previous
import functools

import jax
import jax.numpy as jnp
from jax import lax
from jax.experimental import pallas as pl
from jax.experimental.pallas import tpu as pltpu
from jax.experimental.pallas import tpu_sc as plsc

N = 10000
E = 320000
D = 128

NC = 2
NS = 16
NW = NC * NS

NP = 10240
RPT = NP // NS

EPW = E // NW
CK = 80
DC = EPW // CK
RPT3 = NP // NS
CPY = 80
NBUF = 2

_mesh = plsc.VectorSubcoreMesh(core_axis_name="c", subcore_axis_name="s")


@functools.partial(
    pl.kernel,
    out_type=[jax.ShapeDtypeStruct((NP,), jnp.float32) for _ in range(NC)],
    mesh=_mesh,
    scratch_types=[
        pltpu.VMEM((DC, CK), jnp.int32),
        pltpu.VMEM((128,), jnp.float32),
        pltpu.VMEM((RPT,), jnp.float32),
        pltpu.VMEM_SHARED((NP,), jnp.float32),
    ],
)
def _deg_kernel(dst_hbm, d0_hbm, d1_hbm, idx_v, ones_v, zer_v, deg_sh):
    cid = lax.axis_index("c")
    sid = lax.axis_index("s")
    wid = cid * NS + sid

    for i in range(8):
        ones_v[pl.ds(i * 16, 16)] = jnp.ones((16,), jnp.float32)
    for i in range(RPT // 16):
        zer_v[pl.ds(i * 16, 16)] = jnp.zeros((16,), jnp.float32)

    pltpu.sync_copy(zer_v, deg_sh.at[pl.ds(sid * RPT, RPT)])
    pltpu.sync_copy(dst_hbm.at[wid], idx_v)
    plsc.subcore_barrier()

    def body(j, carry):
        pltpu.sync_copy(ones_v.at[pl.ds(0, CK)], deg_sh.at[idx_v.at[j]], add=True)
        return carry

    lax.fori_loop(0, DC, body, 0)
    plsc.subcore_barrier()

    sl = pl.ds(sid * RPT, RPT)

    @pl.when(cid == 0)
    def _():
        pltpu.sync_copy(deg_sh.at[sl], d0_hbm.at[sl])

    @pl.when(cid == 1)
    def _():
        pltpu.sync_copy(deg_sh.at[sl], d1_hbm.at[sl])


def _mm_body(x_ref, w_ref, d0_ref, d1_ref, h2_ref):
    deg = d0_ref[...] + d1_ref[...] + 1.0
    dis = lax.rsqrt(deg)
    h = jnp.dot(x_ref[...], w_ref[...], preferred_element_type=jnp.float32)
    h2_ref[...] = h * dis


_BN = 400


def _matmul_scale(x, W, d0, d1):
    grid = (N // _BN,)
    return pl.pallas_call(
        _mm_body,
        grid=grid,
        in_specs=[
            pl.BlockSpec((_BN, D), lambda i: (i, 0)),
            pl.BlockSpec((D, D), lambda i: (0, 0)),
            pl.BlockSpec((_BN, 1), lambda i: (i, 0)),
            pl.BlockSpec((_BN, 1), lambda i: (i, 0)),
        ],
        out_specs=pl.BlockSpec((_BN, D), lambda i: (i, 0)),
        out_shape=jax.ShapeDtypeStruct((N, D), jnp.float32),
    )(x, W, d0, d1)


@functools.partial(
    pl.kernel,
    out_type=jax.ShapeDtypeStruct((NC, NP, D), jnp.float32),
    mesh=_mesh,
    scratch_types=[
        pltpu.VMEM((EPW,), jnp.int32),
        pltpu.VMEM((DC, CK), jnp.int32),
        pltpu.VMEM((CK, D), jnp.float32),
        pltpu.VMEM((CK, D), jnp.float32),
        pltpu.VMEM_SHARED((NP, D), jnp.float32),
        pltpu.SemaphoreType.DMA,
        pltpu.SemaphoreType.DMA,
        pltpu.SemaphoreType.DMA,
        pltpu.SemaphoreType.DMA,
    ],
)
def _agg_kernel(src_hbm, dst_hbm, h2_hbm, t_hbm, si_v, di_v, rows0, rows1,
                t_sh, sem0, sem1, sem2, sem3):
    cid = lax.axis_index("c")
    sid = lax.axis_index("s")
    wid = cid * NS + sid

    def zrow(r, carry):
        for c in range(D // 16):
            rows0[r, pl.ds(c * 16, 16)] = jnp.zeros((16,), jnp.float32)
        return carry

    lax.fori_loop(0, CPY, zrow, 0)

    def zcopy(k, carry):
        pltpu.sync_copy(rows0, t_sh.at[pl.ds(sid * RPT3 + k * CPY, CPY)])
        return carry

    lax.fori_loop(0, RPT3 // CPY, zcopy, 0)

    pltpu.sync_copy(src_hbm.at[wid], si_v)
    pltpu.sync_copy(dst_hbm.at[wid], di_v)
    plsc.subcore_barrier()

    def sidx(j):
        return si_v.at[pl.ds(pl.multiple_of(j * CK, CK), CK)]

    pltpu.async_copy(h2_hbm.at[sidx(0)], rows0, sem0).wait()

    def body(i, carry):
        g = i * 2
        pltpu.async_copy(rows0, t_sh.at[di_v.at[g]], sem2, add=True)
        pltpu.async_copy(h2_hbm.at[sidx(g + 1)], rows1, sem1).wait()
        pltpu.make_async_copy(rows0, t_sh.at[di_v.at[g]], sem2).wait()
        pltpu.async_copy(rows1, t_sh.at[di_v.at[g + 1]], sem3, add=True)
        pltpu.async_copy(h2_hbm.at[sidx(g + 2)], rows0, sem0).wait()
        pltpu.make_async_copy(rows1, t_sh.at[di_v.at[g + 1]], sem3).wait()
        return carry

    lax.fori_loop(0, (DC - 1) // 2, body, 0)
    pltpu.sync_copy(rows0, t_sh.at[di_v.at[DC - 1]], add=True)
    plsc.subcore_barrier()

    def ocopy(k, carry):
        sl = pl.ds(sid * RPT3 + k * CPY, CPY)
        pltpu.sync_copy(t_sh.at[sl], t_hbm.at[cid, sl])
        return carry

    lax.fori_loop(0, RPT3 // CPY, ocopy, 0)


def _fin_body(t_ref, h2_ref, d0_ref, d1_ref, b_ref, o_ref):
    deg = d0_ref[...] + d1_ref[...] + 1.0
    dis = lax.rsqrt(deg)
    s = t_ref[0] + t_ref[1] + h2_ref[...]
    o_ref[...] = dis * s + b_ref[...]


def _finalize(t, h2, d0, d1, b):
    grid = (N // _BN,)
    return pl.pallas_call(
        _fin_body,
        grid=grid,
        in_specs=[
            pl.BlockSpec((NC, _BN, D), lambda i: (0, i, 0)),
            pl.BlockSpec((_BN, D), lambda i: (i, 0)),
            pl.BlockSpec((_BN, 1), lambda i: (i, 0)),
            pl.BlockSpec((_BN, 1), lambda i: (i, 0)),
            pl.BlockSpec((1, D), lambda i: (0, 0)),
        ],
        out_specs=pl.BlockSpec((_BN, D), lambda i: (i, 0)),
        out_shape=jax.ShapeDtypeStruct((N, D), jnp.float32),
    )(t, h2, d0, d1, b)


def kernel(x, edge_index, W, b):
    src = edge_index[0].reshape(NW, EPW)
    dst = edge_index[1].reshape(NW, DC, CK)
    d0, d1 = _deg_kernel(dst)
    d0c = d0.reshape(NP, 1)
    d1c = d1.reshape(NP, 1)
    h2 = _matmul_scale(x, W, d0c, d1c)
    t = _agg_kernel(src, dst, h2)
    return _finalize(t, h2, d0c, d1c, b.reshape(1, D))


# speedup vs baseline: 37.8579x; 1.1746x over previous
import functools

import jax
import jax.numpy as jnp
from jax import lax
from jax.experimental import pallas as pl
from jax.experimental.pallas import tpu as pltpu
from jax.experimental.pallas import tpu_sc as plsc

N = 10000
E = 320000
D = 128

NC = 2
NS = 16
NW = NC * NS

NP = 10240
RPT = NP // NS

EPW = E // NW
CK = 80
DC = EPW // CK
RPT3 = NP // NS
CPY = 80
NBUF = 2

_mesh = plsc.VectorSubcoreMesh(core_axis_name="c", subcore_axis_name="s")


@functools.partial(
    pl.kernel,
    out_type=[jax.ShapeDtypeStruct((NP,), jnp.float32) for _ in range(NC)],
    mesh=_mesh,
    scratch_types=[
        pltpu.VMEM((DC, CK), jnp.int32),
        pltpu.VMEM((128,), jnp.float32),
        pltpu.VMEM((RPT,), jnp.float32),
        pltpu.VMEM_SHARED((NP,), jnp.float32),
    ],
)
def _deg_kernel(dst_hbm, d0_hbm, d1_hbm, idx_v, ones_v, zer_v, deg_sh):
    cid = lax.axis_index("c")
    sid = lax.axis_index("s")
    wid = cid * NS + sid

    for i in range(8):
        ones_v[pl.ds(i * 16, 16)] = jnp.ones((16,), jnp.float32)
    for i in range(RPT // 16):
        zer_v[pl.ds(i * 16, 16)] = jnp.zeros((16,), jnp.float32)

    pltpu.sync_copy(zer_v, deg_sh.at[pl.ds(sid * RPT, RPT)])
    pltpu.sync_copy(dst_hbm.at[wid], idx_v)
    plsc.subcore_barrier()

    def body(j, carry):
        pltpu.sync_copy(ones_v.at[pl.ds(0, CK)], deg_sh.at[idx_v.at[j]], add=True)
        return carry

    lax.fori_loop(0, DC, body, 0)
    plsc.subcore_barrier()

    sl = pl.ds(sid * RPT, RPT)

    @pl.when(cid == 0)
    def _():
        pltpu.sync_copy(deg_sh.at[sl], d0_hbm.at[sl])

    @pl.when(cid == 1)
    def _():
        pltpu.sync_copy(deg_sh.at[sl], d1_hbm.at[sl])


def _mm_body(x_ref, w_ref, d0_ref, d1_ref, h2_ref):
    deg = d0_ref[...] + d1_ref[...] + 1.0
    dis = lax.rsqrt(deg)
    h = jnp.dot(x_ref[...], w_ref[...], preferred_element_type=jnp.float32)
    h2_ref[...] = h * dis


_BN = 400


def _matmul_scale(x, W, d0, d1):
    grid = (N // _BN,)
    return pl.pallas_call(
        _mm_body,
        grid=grid,
        in_specs=[
            pl.BlockSpec((_BN, D), lambda i: (i, 0)),
            pl.BlockSpec((D, D), lambda i: (0, 0)),
            pl.BlockSpec((_BN, 1), lambda i: (i, 0)),
            pl.BlockSpec((_BN, 1), lambda i: (i, 0)),
        ],
        out_specs=pl.BlockSpec((_BN, D), lambda i: (i, 0)),
        out_shape=jax.ShapeDtypeStruct((N, D), jnp.float32),
    )(x, W, d0, d1)


@functools.partial(
    pl.kernel,
    out_type=jax.ShapeDtypeStruct((NC, NP, D), jnp.float32),
    mesh=_mesh,
    scratch_types=[
        pltpu.VMEM((EPW,), jnp.int32),
        pltpu.VMEM((DC, CK), jnp.int32),
        pltpu.VMEM((CK, D), jnp.float32),
        pltpu.VMEM((CK, D), jnp.float32),
        pltpu.VMEM_SHARED((NP, D), jnp.float32),
        pltpu.SemaphoreType.DMA,
        pltpu.SemaphoreType.DMA,
        pltpu.SemaphoreType.DMA,
        pltpu.SemaphoreType.DMA,
    ],
)
def _agg_kernel(src_hbm, dst_hbm, h2_hbm, t_hbm, si_v, di_v, rows0, rows1,
                t_sh, sem0, sem1, sem2, sem3):
    cid = lax.axis_index("c")
    sid = lax.axis_index("s")
    wid = cid * NS + sid

    def zrow(r, carry):
        for c in range(D // 16):
            rows0[r, pl.ds(c * 16, 16)] = jnp.zeros((16,), jnp.float32)
        return carry

    lax.fori_loop(0, CPY, zrow, 0)

    def zcopy(k, carry):
        pltpu.sync_copy(rows0, t_sh.at[pl.ds(sid * RPT3 + k * CPY, CPY)])
        return carry

    lax.fori_loop(0, RPT3 // CPY, zcopy, 0)

    pltpu.sync_copy(src_hbm.at[wid], si_v)
    pltpu.sync_copy(dst_hbm.at[wid], di_v)
    plsc.subcore_barrier()

    def sidx(j):
        return si_v.at[pl.ds(j * CK, CK)]

    hg = [None, None]
    hs = [None, None]
    rows = [rows0, rows1]
    semg = [sem0, sem1]
    sems = [sem2, sem3]
    for j in range(DC):
        bf = j % 2
        if hs[bf] is not None:
            hs[bf].wait()
        hg[bf] = pltpu.async_copy(h2_hbm.at[sidx(j)], rows[bf], semg[bf])
        k = j - 1
        if k >= 0:
            bk = k % 2
            hg[bk].wait()
            hs[bk] = pltpu.async_copy(rows[bk], t_sh.at[di_v.at[k]], sems[bk],
                                      add=True)
    bl = (DC - 1) % 2
    hg[bl].wait()
    hs[bl] = pltpu.async_copy(rows[bl], t_sh.at[di_v.at[DC - 1]], sems[bl],
                              add=True)
    hs[0].wait()
    hs[1].wait()
    plsc.subcore_barrier()

    def ocopy(k, carry):
        sl = pl.ds(sid * RPT3 + k * CPY, CPY)
        pltpu.sync_copy(t_sh.at[sl], t_hbm.at[cid, sl])
        return carry

    lax.fori_loop(0, RPT3 // CPY, ocopy, 0)


def _fin_body(t_ref, h2_ref, d0_ref, d1_ref, b_ref, o_ref):
    deg = d0_ref[...] + d1_ref[...] + 1.0
    dis = lax.rsqrt(deg)
    s = t_ref[0] + t_ref[1] + h2_ref[...]
    o_ref[...] = dis * s + b_ref[...]


def _finalize(t, h2, d0, d1, b):
    grid = (N // _BN,)
    return pl.pallas_call(
        _fin_body,
        grid=grid,
        in_specs=[
            pl.BlockSpec((NC, _BN, D), lambda i: (0, i, 0)),
            pl.BlockSpec((_BN, D), lambda i: (i, 0)),
            pl.BlockSpec((_BN, 1), lambda i: (i, 0)),
            pl.BlockSpec((_BN, 1), lambda i: (i, 0)),
            pl.BlockSpec((1, D), lambda i: (0, 0)),
        ],
        out_specs=pl.BlockSpec((_BN, D), lambda i: (i, 0)),
        out_shape=jax.ShapeDtypeStruct((N, D), jnp.float32),
    )(t, h2, d0, d1, b)


def kernel(x, edge_index, W, b):
    src = edge_index[0].reshape(NW, EPW)
    dst = edge_index[1].reshape(NW, DC, CK)
    d0, d1 = _deg_kernel(dst)
    d0c = d0.reshape(NP, 1)
    d1c = d1.reshape(NP, 1)
    h2 = _matmul_scale(x, W, d0c, d1c)
    t = _agg_kernel(src, dst, h2)
    return _finalize(t, h2, d0c, d1c, b.reshape(1, D))
